# baseline (speedup 1.0000x reference)
import jax
import jax.numpy as jnp
from jax import lax
from jax.experimental import pallas as pl
from jax.experimental.pallas import tpu as pltpu
from jax.experimental.pallas import tpu_sc as plsc

_R = 512
_N = 2048
_BPW = _N // _R

_SC_TILES = 16
_SC_WROWS = _N // (2 * _SC_TILES)
_SC_CHUNK = 32
_SC_ITERS = _SC_WROWS // _SC_CHUNK


def _tc_body(x_ref, w1_ref, w2_ref, w3_ref, b1_ref, b2_ref, b3_ref,
             mw_ref, mb_ref, w1o_ref, out_ref, h1_ref, h2_ref):
    l = pl.program_id(0)
    t = pl.program_id(1)

    def compute(w_ref, b_ref, vec):
        y = jax.lax.dot_general(
            vec, w_ref[...], (((1,), (1,)), ((), ())),
            preferred_element_type=jnp.float32,
        )
        return jnp.maximum(y + b_ref[0:1, pl.ds(t * _R, _R)], 0.0)

    @pl.when(l == 0)
    def _layer1():
        vec = x_ref[...]
        h = compute(w1_ref, b1_ref, vec)
        h1_ref[0:1, pl.ds(t * _R, _R)] = h
        w1o_ref[...] = w1_ref[...]

        @pl.when(t == 0)
        def _():
            cols_h = jax.lax.broadcasted_iota(jnp.int32, h.shape, 1)
            h0 = jnp.sum(jnp.where(cols_h == 0, h, 0.0))
            row0 = w1_ref[0:1, :]
            cols_w = jax.lax.broadcasted_iota(jnp.int32, row0.shape, 1)
            w00 = jnp.sum(jnp.where(cols_w == 0, row0, 0.0))
            s = jnp.sum(jnp.where(cols_w == 0, vec, 0.0))
            new00 = (s * mw_ref[0, 0] + w00 * mw_ref[0, 1]
                     + h0 * mw_ref[0, 2] + mb_ref[0])
            w1o_ref[0:1, :] = jnp.where(cols_w == 0, new00, row0)

    @pl.when(l == 1)
    def _layer2():
        h = compute(w2_ref, b2_ref, h1_ref[...])
        h2_ref[0:1, pl.ds(t * _R, _R)] = h

    @pl.when(l == 2)
    def _layer3():
        h = compute(w3_ref, b3_ref, h2_ref[...])
        out_ref[...] = h


def _tc_main(x, W1, W2, W3, b1, b2, b3, meta_W, meta_b):
    last = _BPW - 1
    w_spec = [
        pl.BlockSpec((_R, _N), lambda l, t: (jnp.where(l == 0, t, last), 0)),
        pl.BlockSpec((_R, _N), lambda l, t: (jnp.where(l == 0, 0, jnp.where(l == 1, t, last)), 0)),
        pl.BlockSpec((_R, _N), lambda l, t: (jnp.where(l == 2, t, 0), 0)),
    ]
    return pl.pallas_call(
        _tc_body,
        grid=(3, _BPW),
        in_specs=[
            pl.BlockSpec((1, _N), lambda l, t: (0, 0)),
            w_spec[0], w_spec[1], w_spec[2],
            pl.BlockSpec((1, _N), lambda l, t: (0, 0)),
            pl.BlockSpec((1, _N), lambda l, t: (0, 0)),
            pl.BlockSpec((1, _N), lambda l, t: (0, 0)),
            pl.BlockSpec(memory_space=pltpu.SMEM),
            pl.BlockSpec(memory_space=pltpu.SMEM),
        ],
        out_specs=[
            w_spec[0],
            pl.BlockSpec((1, _R), lambda l, t: (0, jnp.where(l == 2, t, 0))),
            pl.BlockSpec((1, _N), lambda l, t: (0, 0)),
            pl.BlockSpec((1, _N), lambda l, t: (0, 0)),
        ],
        out_shape=[
            jax.ShapeDtypeStruct((_N, _N), jnp.float32),
            jax.ShapeDtypeStruct((1, _N), jnp.float32),
            jax.ShapeDtypeStruct((1, _N), jnp.float32),
            jax.ShapeDtypeStruct((1, _N), jnp.float32),
        ],
    )(x, W1, W2, W3, b1.reshape(1, -1), b2.reshape(1, -1),
      b3.reshape(1, -1), meta_W, meta_b)


def _sc_body(w2_hbm, w3_hbm, o2_hbm, o3_hbm, vbuf):
    c = lax.axis_index("c")
    s = lax.axis_index("s")
    wid = c * _SC_TILES + s
    base = wid * _SC_WROWS
    for j in range(_SC_ITERS):
        row0 = base + j * _SC_CHUNK
        pltpu.sync_copy(w2_hbm.at[pl.ds(row0, _SC_CHUNK), :], vbuf)
        pltpu.sync_copy(vbuf, o2_hbm.at[pl.ds(row0, _SC_CHUNK), :])
        pltpu.sync_copy(w3_hbm.at[pl.ds(row0, _SC_CHUNK), :], vbuf)
        pltpu.sync_copy(vbuf, o3_hbm.at[pl.ds(row0, _SC_CHUNK), :])


def _sc_copy(W2, W3):
    mesh = plsc.VectorSubcoreMesh(core_axis_name="c", subcore_axis_name="s")
    return pl.kernel(
        _sc_body,
        out_type=[
            jax.ShapeDtypeStruct((_N, _N), jnp.float32),
            jax.ShapeDtypeStruct((_N, _N), jnp.float32),
        ],
        mesh=mesh,
        scratch_types=[pltpu.VMEM((_SC_CHUNK, _N), jnp.float32)],
    )(W2, W3)


def _fix_body(w2r_ref, w3r_ref, h1_ref, h2_ref, out_ref, mw_ref, mb_ref,
              o2_ref, o3_ref):
    def patch(w_ref, o_ref, vec_ref, ovec_ref):
        blk = w_ref[...]
        rows = jax.lax.broadcasted_iota(jnp.int32, blk.shape, 0)
        cols = jax.lax.broadcasted_iota(jnp.int32, blk.shape, 1)
        m00 = (rows == 0) & (cols == 0)
        w00 = jnp.sum(jnp.where(m00, blk, 0.0))
        cols1 = jax.lax.broadcasted_iota(jnp.int32, (1, _N), 1)
        s = jnp.sum(jnp.where(cols1 == 0, vec_ref[...], 0.0))
        o0 = jnp.sum(jnp.where(cols1 == 0, ovec_ref[...], 0.0))
        new00 = (s * mw_ref[0, 0] + w00 * mw_ref[0, 1]
                 + o0 * mw_ref[0, 2] + mb_ref[0])
        o_ref[...] = jnp.where(m00, new00, blk)

    patch(w2r_ref, o2_ref, h1_ref, h2_ref)
    patch(w3r_ref, o3_ref, h2_ref, out_ref)


def _fix(W2c, W3c, h1, h2, out, meta_W, meta_b):
    blk8 = pl.BlockSpec((8, _N), lambda i: (0, 0))
    vec = pl.BlockSpec((1, _N), lambda i: (0, 0))
    return pl.pallas_call(
        _fix_body,
        grid=(1,),
        in_specs=[blk8, blk8, vec, vec, vec,
                  pl.BlockSpec(memory_space=pltpu.SMEM),
                  pl.BlockSpec(memory_space=pltpu.SMEM)],
        out_specs=[blk8, blk8],
        out_shape=[
            jax.ShapeDtypeStruct((_N, _N), jnp.float32),
            jax.ShapeDtypeStruct((_N, _N), jnp.float32),
        ],
        input_output_aliases={0: 0, 1: 1},
    )(W2c, W3c, h1, h2, out, meta_W, meta_b)


def kernel(x, W1, b1, W2, b2, W3, b3, meta_W, meta_b):
    W2c, W3c = _sc_copy(W2, W3)
    W1n, out, h1, h2 = _tc_main(x, W1, W2, W3, b1, b2, b3, meta_W, meta_b)
    W2n, W3n = _fix(W2c, W3c, h1, h2, out, meta_W, meta_b)
    return (out, W1n, W2n, W3n)


# speedup vs baseline: 1.6311x; 1.6311x over previous
import jax
import jax.numpy as jnp
from jax.experimental import pallas as pl
from jax.experimental.pallas import tpu as pltpu

_R = 512
_N = 2048
_BPW = _N // _R


def _body(x_ref, w1_ref, w2_ref, w3_ref, b1_ref, b2_ref, b3_ref,
          mw_ref, mb_ref, w1o_ref, w2o_ref, w3o_ref, out_ref,
          h1_ref, h2_ref):
    l = pl.program_id(0)
    t = pl.program_id(1)

    def compute(w_ref, b_ref, vec):
        y = jax.lax.dot_general(
            vec, w_ref[...], (((1,), (1,)), ((), ())),
            preferred_element_type=jnp.float32,
        )
        h = jnp.maximum(y + b_ref[0:1, pl.ds(t * _R, _R)], 0.0)
        return h

    def fixup(w_ref, w_out_ref, vec, h):
        cols_h = jax.lax.broadcasted_iota(jnp.int32, h.shape, 1)
        h0 = jnp.sum(jnp.where(cols_h == 0, h, 0.0))
        row0 = w_ref[0:1, :]
        cols_w = jax.lax.broadcasted_iota(jnp.int32, row0.shape, 1)
        w00 = jnp.sum(jnp.where(cols_w == 0, row0, 0.0))
        s = jnp.sum(jnp.where(cols_w == 0, vec, 0.0))
        new00 = (s * mw_ref[0, 0] + w00 * mw_ref[0, 1]
                 + h0 * mw_ref[0, 2] + mb_ref[0])
        w_out_ref[0:1, :] = jnp.where(cols_w == 0, new00, row0)

    @pl.when(l == 0)
    def _layer1():
        vec = x_ref[...]
        h = compute(w1_ref, b1_ref, vec)
        h1_ref[0:1, pl.ds(t * _R, _R)] = h
        w1o_ref[...] = w1_ref[...]

        @pl.when(t == 0)
        def _():
            fixup(w1_ref, w1o_ref, vec, h)

    @pl.when(l == 1)
    def _layer2():
        vec = h1_ref[...]
        h = compute(w2_ref, b2_ref, vec)
        h2_ref[0:1, pl.ds(t * _R, _R)] = h
        w2o_ref[...] = w2_ref[...]

        @pl.when(t == 0)
        def _():
            fixup(w2_ref, w2o_ref, vec, h)

    @pl.when(l == 2)
    def _layer3():
        vec = h2_ref[...]
        h = compute(w3_ref, b3_ref, vec)
        out_ref[...] = h
        w3o_ref[...] = w3_ref[...]

        @pl.when(t == 0)
        def _():
            fixup(w3_ref, w3o_ref, vec, h)


def kernel(x, W1, b1, W2, b2, W3, b3, meta_W, meta_b):
    last = _BPW - 1
    w_spec = [
        pl.BlockSpec((_R, _N), lambda l, t: (jnp.where(l == 0, t, last), 0)),
        pl.BlockSpec((_R, _N), lambda l, t: (jnp.where(l == 0, 0, jnp.where(l == 1, t, last)), 0)),
        pl.BlockSpec((_R, _N), lambda l, t: (jnp.where(l == 2, t, 0), 0)),
    ]
    W1n, W2n, W3n, out = pl.pallas_call(
        _body,
        grid=(3, _BPW),
        in_specs=[
            pl.BlockSpec((1, _N), lambda l, t: (0, 0)),
            w_spec[0], w_spec[1], w_spec[2],
            pl.BlockSpec((1, _N), lambda l, t: (0, 0)),
            pl.BlockSpec((1, _N), lambda l, t: (0, 0)),
            pl.BlockSpec((1, _N), lambda l, t: (0, 0)),
            pl.BlockSpec(memory_space=pltpu.SMEM),
            pl.BlockSpec(memory_space=pltpu.SMEM),
        ],
        out_specs=[
            w_spec[0], w_spec[1], w_spec[2],
            pl.BlockSpec((1, _R), lambda l, t: (0, jnp.where(l == 2, t, 0))),
        ],
        out_shape=[
            jax.ShapeDtypeStruct((_N, _N), jnp.float32),
            jax.ShapeDtypeStruct((_N, _N), jnp.float32),
            jax.ShapeDtypeStruct((_N, _N), jnp.float32),
            jax.ShapeDtypeStruct((1, _N), jnp.float32),
        ],
        scratch_shapes=[
            pltpu.VMEM((1, _N), jnp.float32),
            pltpu.VMEM((1, _N), jnp.float32),
        ],
    )(x, W1, W2, W3, b1.reshape(1, -1), b2.reshape(1, -1),
      b3.reshape(1, -1), meta_W, meta_b)
    return (out, W1n, W2n, W3n)
